# Initial kernel scaffold; baseline (speedup 1.0000x reference)
import jax
import jax.numpy as jnp
from jax.experimental import pallas as pl


def kernel(predictions, targets, indexes):
    raise NotImplementedError("write your pallas kernel here")


# speedup vs baseline: 62.8141x; 62.8141x over previous
import functools
import numpy as np
import jax
import jax.numpy as jnp
from jax import lax
from jax.experimental import pallas as pl
from jax.experimental.pallas import tpu as pltpu
from jax.experimental.pallas import tpu_sc as plsc

K_TOP = 10
NQ = 16384
NELEM = 819200
NC = 2
NS = 16
NW = NC * NS
CH = NELEM // NW
GPW = NQ // NW
W = 2048
PADW = W + 16

_DISC = np.zeros((16,), np.float32)
_DISC[:K_TOP] = 1.0 / np.log2(np.arange(K_TOP) + 2.0)

_MESH = dict(core_axis_name="c", subcore_axis_name="s", num_cores=NC,
             num_subcores=NS)


@functools.partial(
    pl.kernel,
    out_type=jax.ShapeDtypeStruct((NW, NQ), jnp.int32),
    mesh=plsc.VectorSubcoreMesh(**_MESH),
    scratch_types=[
        pltpu.VMEM((CH + 8,), jnp.int32),
        pltpu.VMEM((NQ,), jnp.int32),
    ],
    compiler_params=pltpu.CompilerParams(needs_layout_passes=False),
)
def _phase_a(idx_hbm, out_hbm, buf, fp):
    wid = lax.axis_index("s") * NC + lax.axis_index("c")
    pltpu.sync_copy(idx_hbm.at[pl.ds(wid * CH, CH + 8)], buf)

    filln = jnp.full((16,), NELEM, jnp.int32)

    def init_body(i, c):
        fp[pl.ds(i * 16, 16)] = filln
        return c

    lax.fori_loop(0, NQ // 16, init_body, 0)

    iot = lax.iota(jnp.int32, 16)
    base_pos = wid * CH

    def scan_body(i, c):
        v = buf[pl.ds(i * 16 + 8, 16)]
        vp = buf[pl.ds(i * 16 + 7, 16)]
        m = v != vp
        pos = iot + (base_pos + i * 16)
        plsc.store_scatter(fp, [v], pos, mask=m)
        return c

    lax.fori_loop(0, CH // 16, scan_body, 0)

    pltpu.sync_copy(fp, out_hbm.at[wid])


@functools.partial(
    pl.kernel,
    out_type=jax.ShapeDtypeStruct((NW, 16), jnp.float32),
    mesh=plsc.VectorSubcoreMesh(**_MESH),
    scratch_types=[
        pltpu.VMEM((GPW + 16,), jnp.int32),
        pltpu.VMEM((GPW,), jnp.int32),
        pltpu.VMEM((W,), jnp.int32),
        pltpu.VMEM((W,), jnp.float32),
        pltpu.VMEM((W,), jnp.float32),
        pltpu.VMEM((16,), jnp.float32),
        pltpu.VMEM((16,), jnp.float32),
    ],
    compiler_params=pltpu.CompilerParams(needs_layout_passes=False),
)
def _phase_b(fp_hbm, idx_hbm, prd_hbm, tgt_hbm, disc_hbm, out_hbm,
             minbuf, rowbuf, idxw, prdw, tgtw, discb, outb):
    wid = lax.axis_index("s") * NC + lax.axis_index("c")
    col0 = wid * GPW

    pltpu.sync_copy(disc_hbm, discb)
    disc = discb[...]

    pltpu.sync_copy(fp_hbm.at[0, pl.ds(col0, GPW)], minbuf.at[pl.ds(0, GPW)])

    def row_body(r, c):
        pltpu.sync_copy(fp_hbm.at[r, pl.ds(col0, GPW)], rowbuf)

        def min_body(k, c2):
            sl = pl.ds(k * 16, 16)
            minbuf[sl] = jnp.minimum(minbuf[sl], rowbuf[sl])
            return c2

        lax.fori_loop(0, GPW // 16, min_body, 0)
        return c

    lax.fori_loop(1, NW, row_body, 0)

    neg1 = jnp.full((16,), -1.0, jnp.float32)
    zero = jnp.zeros((16,), jnp.float32)
    iot = lax.iota(jnp.int32, 16)

    def group_body(j, carry):
        base, snd, prs = carry
        s = minbuf[pl.ds(j, 16)][0]
        g = col0 + j

        def cond_fn(st):
            return st[0]

        def body_fn(st):
            _, p, b, ak, av, ai = st
            need = p + 16 > b + W

            def reload(_):
                nb = (p // 8) * 8
                pltpu.sync_copy(idx_hbm.at[pl.ds(nb, W)], idxw)
                pltpu.sync_copy(prd_hbm.at[pl.ds(nb, W)], prdw)
                pltpu.sync_copy(tgt_hbm.at[pl.ds(nb, W)], tgtw)
                return nb

            b = lax.cond(need, reload, lambda _: b, 0)
            off = p - b
            vi = idxw[pl.ds(off, 16)]
            pv = prdw[pl.ds(off, 16)]
            tv = tgtw[pl.ds(off, 16)]
            valid = vi == g
            ck = jnp.where(valid, pv, -1.0)
            cv = jnp.where(valid, tv, 0.0)
            sk, sv = plsc.sort_key_val(ck, cv, descending=True)
            rk = lax.rev(sk, (0,))
            rv = lax.rev(sv, (0,))
            ta = ak >= rk
            ak, av = plsc.sort_key_val(
                jnp.where(ta, ak, rk), jnp.where(ta, av, rv), descending=True)
            ki = jnp.where(valid, tv, -1.0)
            si, _ = plsc.sort_key_val(ki, ki, descending=True)
            mi = jnp.maximum(ai, lax.rev(si, (0,)))
            ai, _ = plsc.sort_key_val(mi, mi, descending=True)
            nval = jnp.sum(valid.astype(jnp.int32))
            return (nval == 16, p + 16, b, ak, av, ai)

        cont0 = s < NELEM
        st = (cont0, s, base, neg1, zero, neg1)
        st = lax.while_loop(cond_fn, body_fn, st)
        _, _, base, ak, av, ai = st
        dcg_v = lax.cumsum(av * disc, axis=0)
        idcg_v = lax.cumsum(jnp.maximum(ai, 0.0) * disc, axis=0)
        safe_v = jnp.where(idcg_v > 0.0, idcg_v, 1.0)
        nd_v = jnp.where(idcg_v > 0.0, dcg_v / safe_v, 0.0)
        snd = snd + nd_v[15]
        prs = prs + jnp.where(cont0, 1.0, 0.0)
        return (base, snd, prs)

    base0 = jnp.int32(-2 * W)
    _, snd, prs = lax.fori_loop(
        0, GPW, group_body, (base0, jnp.float32(0.0), jnp.float32(0.0)))

    outb[...] = jnp.where(iot == 0, snd, jnp.where(iot == 1, prs, 0.0))
    pltpu.sync_copy(outb, out_hbm.at[wid])


_BCE_ROWS = 6400
_BCE_BR = 256


def _bce_body(p_ref, t_ref, o_ref):
    i = pl.program_id(0)
    eps = 1e-7
    p = jnp.clip(p_ref[...], eps, 1.0 - eps)
    t = t_ref[...]
    s = jnp.sum(t * jnp.log(p) + (1.0 - t) * jnp.log(1.0 - p))

    @pl.when(i == 0)
    def _():
        o_ref[0, 0] = s

    @pl.when(i > 0)
    def _():
        o_ref[0, 0] = o_ref[0, 0] + s


def _bce_tc(p2, t2):
    return pl.pallas_call(
        _bce_body,
        grid=(_BCE_ROWS // _BCE_BR,),
        in_specs=[
            pl.BlockSpec((_BCE_BR, 128), lambda i: (i, 0)),
            pl.BlockSpec((_BCE_BR, 128), lambda i: (i, 0)),
        ],
        out_specs=pl.BlockSpec(memory_space=pltpu.SMEM),
        out_shape=jax.ShapeDtypeStruct((1, 1), jnp.float32),
    )(p2, t2)


@jax.jit
def kernel(predictions, targets, indexes):
    idx = indexes.astype(jnp.int32)
    idx_lead = jnp.concatenate([jnp.full((8,), -1, jnp.int32), idx])
    fp = _phase_a(idx_lead)
    idx_pad = jnp.concatenate([idx, jnp.full((PADW,), NQ, jnp.int32)])
    prd_pad = jnp.concatenate([predictions, jnp.zeros((PADW,), jnp.float32)])
    tgt_pad = jnp.concatenate([targets, jnp.zeros((PADW,), jnp.float32)])
    parts = _phase_b(fp, idx_pad, prd_pad, tgt_pad, jnp.asarray(_DISC))
    bce = _bce_tc(predictions.reshape(_BCE_ROWS, 128),
                  targets.reshape(_BCE_ROWS, 128))[0, 0]
    xe = -bce / NELEM
    snd = jnp.sum(parts[:, 0])
    prs = jnp.sum(parts[:, 1])
    ndcg = snd / jnp.maximum(prs, 1.0)
    return xe * (1.0 - ndcg)


# speedup vs baseline: 80.3825x; 1.2797x over previous
import functools
import numpy as np
import jax
import jax.numpy as jnp
from jax import lax
from jax.experimental import pallas as pl
from jax.experimental.pallas import tpu as pltpu
from jax.experimental.pallas import tpu_sc as plsc

K_TOP = 10
NQ = 16384
NELEM = 819200
NC = 2
NS = 16
NW = NC * NS
CH = NELEM // NW
GPW = NQ // NW
W = 8192
PADW = W + 32

_DISC = np.zeros((16,), np.float32)
_DISC[:K_TOP] = 1.0 / np.log2(np.arange(K_TOP) + 2.0)

_MESH = dict(core_axis_name="c", subcore_axis_name="s", num_cores=NC,
             num_subcores=NS)


@functools.partial(
    pl.kernel,
    out_type=jax.ShapeDtypeStruct((NW, NQ), jnp.int32),
    mesh=plsc.VectorSubcoreMesh(**_MESH),
    scratch_types=[
        pltpu.VMEM((CH + 8,), jnp.int32),
        pltpu.VMEM((NQ,), jnp.int32),
    ],
    compiler_params=pltpu.CompilerParams(needs_layout_passes=False),
)
def _phase_a(idx_hbm, out_hbm, buf, fp):
    wid = lax.axis_index("s") * NC + lax.axis_index("c")
    pltpu.sync_copy(idx_hbm.at[pl.ds(wid * CH, CH + 8)], buf)

    filln = jnp.full((16,), NELEM, jnp.int32)

    def init_body(i, c):
        fp[pl.ds(i * 16, 16)] = filln
        return c

    lax.fori_loop(0, NQ // 16, init_body, 0)

    iot = lax.iota(jnp.int32, 16)
    base_pos = wid * CH

    def scan_body(i, c):
        v = buf[pl.ds(i * 16 + 8, 16)]
        vp = buf[pl.ds(i * 16 + 7, 16)]
        m = v != vp
        pos = iot + (base_pos + i * 16)
        plsc.store_scatter(fp, [v], pos, mask=m)
        return c

    lax.fori_loop(0, CH // 16, scan_body, 0)

    pltpu.sync_copy(fp, out_hbm.at[wid])


@functools.partial(
    pl.kernel,
    out_type=jax.ShapeDtypeStruct((NW, 16), jnp.float32),
    mesh=plsc.VectorSubcoreMesh(**_MESH),
    scratch_types=[
        pltpu.VMEM((GPW + 16,), jnp.int32),
        pltpu.VMEM((NW, GPW), jnp.int32),
        pltpu.VMEM((W,), jnp.int32),
        pltpu.VMEM((W,), jnp.float32),
        pltpu.VMEM((W,), jnp.float32),
        pltpu.VMEM((16,), jnp.float32),
        pltpu.VMEM((16,), jnp.float32),
    ],
    compiler_params=pltpu.CompilerParams(needs_layout_passes=False),
)
def _phase_b(fp_hbm, idx_hbm, prd_hbm, tgt_hbm, disc_hbm, out_hbm,
             minbuf, rowbuf, idxw, prdw, tgtw, discb, outb):
    wid = lax.axis_index("s") * NC + lax.axis_index("c")
    col0 = wid * GPW

    pltpu.sync_copy(disc_hbm, discb)
    disc = discb[...]

    pltpu.sync_copy(fp_hbm.at[:, pl.ds(col0, GPW)], rowbuf)

    def min_outer(k, c):
        sl = pl.ds(k * 16, 16)

        def min_inner(r, m):
            return jnp.minimum(m, rowbuf[r, sl])

        minbuf[sl] = lax.fori_loop(1, NW, min_inner, rowbuf[0, sl])
        return c

    lax.fori_loop(0, GPW // 16, min_outer, 0)

    neg1 = jnp.full((16,), -1.0, jnp.float32)
    zero = jnp.zeros((16,), jnp.float32)
    iot = lax.iota(jnp.int32, 16)

    def group_body(j, carry):
        base, snd, prs = carry
        s = minbuf[pl.ds(j, 16)][0]
        g = col0 + j

        def cond_fn(st):
            return st[0]

        def body_fn(st):
            _, p, b, ak, av, ai = st
            need = p + 32 > b + W

            def reload(_):
                nb = (p // 8) * 8
                pltpu.sync_copy(idx_hbm.at[pl.ds(nb, W)], idxw)
                pltpu.sync_copy(prd_hbm.at[pl.ds(nb, W)], prdw)
                pltpu.sync_copy(tgt_hbm.at[pl.ds(nb, W)], tgtw)
                return nb

            b = lax.cond(need, reload, lambda _: b, 0)
            off = p - b
            vi1 = idxw[pl.ds(off, 16)]
            vi2 = idxw[pl.ds(off + 16, 16)]
            pv1 = prdw[pl.ds(off, 16)]
            pv2 = prdw[pl.ds(off + 16, 16)]
            tv1 = tgtw[pl.ds(off, 16)]
            tv2 = tgtw[pl.ds(off + 16, 16)]
            va1 = vi1 == g
            va2 = vi2 == g
            s1k, s1v = plsc.sort_key_val(
                jnp.where(va1, pv1, -1.0), jnp.where(va1, tv1, 0.0),
                descending=True)
            s2k, s2v = plsc.sort_key_val(
                jnp.where(va2, pv2, -1.0), jnp.where(va2, tv2, 0.0),
                descending=True)
            r2k = lax.rev(s2k, (0,))
            r2v = lax.rev(s2v, (0,))
            tc = s1k >= r2k
            ck, cv = plsc.sort_key_val(
                jnp.where(tc, s1k, r2k), jnp.where(tc, s1v, r2v),
                descending=True)
            rk = lax.rev(ck, (0,))
            rv = lax.rev(cv, (0,))
            ta = ak >= rk
            ak, av = plsc.sort_key_val(
                jnp.where(ta, ak, rk), jnp.where(ta, av, rv), descending=True)
            ki1 = jnp.where(va1, tv1, -1.0)
            ki2 = jnp.where(va2, tv2, -1.0)
            i1, _ = plsc.sort_key_val(ki1, ki1, descending=True)
            i2, _ = plsc.sort_key_val(ki2, ki2, descending=True)
            mc = jnp.maximum(i1, lax.rev(i2, (0,)))
            ci, _ = plsc.sort_key_val(mc, mc, descending=True)
            ma = jnp.maximum(ai, lax.rev(ci, (0,)))
            ai, _ = plsc.sort_key_val(ma, ma, descending=True)
            return (vi2[15] == g, p + 32, b, ak, av, ai)

        cont0 = s < NELEM
        st = (cont0, s, base, neg1, zero, neg1)
        st = lax.while_loop(cond_fn, body_fn, st)
        _, _, base, ak, av, ai = st
        dcg_v = lax.cumsum(av * disc, axis=0)
        idcg_v = lax.cumsum(jnp.maximum(ai, 0.0) * disc, axis=0)
        safe_v = jnp.where(idcg_v > 0.0, idcg_v, 1.0)
        nd_v = jnp.where(idcg_v > 0.0, dcg_v / safe_v, 0.0)
        snd = snd + nd_v[15]
        prs = prs + jnp.where(cont0, 1.0, 0.0)
        return (base, snd, prs)

    base0 = jnp.int32(-2 * W)
    _, snd, prs = lax.fori_loop(
        0, GPW, group_body, (base0, jnp.float32(0.0), jnp.float32(0.0)))

    outb[...] = jnp.where(iot == 0, snd, jnp.where(iot == 1, prs, 0.0))
    pltpu.sync_copy(outb, out_hbm.at[wid])


_BCE_ROWS = 6400
_BCE_BR = 256


def _bce_body(p_ref, t_ref, o_ref):
    i = pl.program_id(0)
    eps = 1e-7
    p = jnp.clip(p_ref[...], eps, 1.0 - eps)
    t = t_ref[...]
    s = jnp.sum(t * jnp.log(p) + (1.0 - t) * jnp.log(1.0 - p))

    @pl.when(i == 0)
    def _():
        o_ref[0, 0] = s

    @pl.when(i > 0)
    def _():
        o_ref[0, 0] = o_ref[0, 0] + s


def _bce_tc(p2, t2):
    return pl.pallas_call(
        _bce_body,
        grid=(_BCE_ROWS // _BCE_BR,),
        in_specs=[
            pl.BlockSpec((_BCE_BR, 128), lambda i: (i, 0)),
            pl.BlockSpec((_BCE_BR, 128), lambda i: (i, 0)),
        ],
        out_specs=pl.BlockSpec(memory_space=pltpu.SMEM),
        out_shape=jax.ShapeDtypeStruct((1, 1), jnp.float32),
    )(p2, t2)


@jax.jit
def kernel(predictions, targets, indexes):
    idx = indexes.astype(jnp.int32)
    idx_lead = jnp.concatenate([jnp.full((8,), -1, jnp.int32), idx])
    fp = _phase_a(idx_lead)
    idx_pad = jnp.concatenate([idx, jnp.full((PADW,), NQ, jnp.int32)])
    prd_pad = jnp.concatenate([predictions, jnp.zeros((PADW,), jnp.float32)])
    tgt_pad = jnp.concatenate([targets, jnp.zeros((PADW,), jnp.float32)])
    parts = _phase_b(fp, idx_pad, prd_pad, tgt_pad, jnp.asarray(_DISC))
    bce = _bce_tc(predictions.reshape(_BCE_ROWS, 128),
                  targets.reshape(_BCE_ROWS, 128))[0, 0]
    xe = -bce / NELEM
    snd = jnp.sum(parts[:, 0])
    prs = jnp.sum(parts[:, 1])
    ndcg = snd / jnp.maximum(prs, 1.0)
    return xe * (1.0 - ndcg)


# speedup vs baseline: 108.9188x; 1.3550x over previous
import functools
import numpy as np
import jax
import jax.numpy as jnp
from jax import lax
from jax.experimental import pallas as pl
from jax.experimental.pallas import tpu as pltpu
from jax.experimental.pallas import tpu_sc as plsc

K_TOP = 10
NQ = 16384
NELEM = 819200
NC = 2
NS = 16
NW = NC * NS
CH = NELEM // NW
GPW = NQ // NW
W = 8192
PADW = W + 64

_DISC = np.zeros((16,), np.float32)
_DISC[:K_TOP] = 1.0 / np.log2(np.arange(K_TOP) + 2.0)

_MESH = dict(core_axis_name="c", subcore_axis_name="s", num_cores=NC,
             num_subcores=NS)


@functools.partial(
    pl.kernel,
    out_type=jax.ShapeDtypeStruct((NW, NQ), jnp.int32),
    mesh=plsc.VectorSubcoreMesh(**_MESH),
    scratch_types=[
        pltpu.VMEM((CH + 8,), jnp.int32),
        pltpu.VMEM((NQ,), jnp.int32),
    ],
    compiler_params=pltpu.CompilerParams(needs_layout_passes=False),
)
def _phase_a(idx_hbm, out_hbm, buf, fp):
    wid = lax.axis_index("s") * NC + lax.axis_index("c")
    pltpu.sync_copy(idx_hbm.at[pl.ds(wid * CH, CH + 8)], buf)

    filln = jnp.full((16,), NELEM, jnp.int32)

    def init_body(i, c):
        fp[pl.ds(i * 16, 16)] = filln
        return c

    lax.fori_loop(0, NQ // 16, init_body, 0)

    iot = lax.iota(jnp.int32, 16)
    base_pos = wid * CH

    def scan_body(i, c):
        v = buf[pl.ds(i * 16 + 8, 16)]
        vp = buf[pl.ds(i * 16 + 7, 16)]
        m = v != vp
        pos = iot + (base_pos + i * 16)
        plsc.store_scatter(fp, [v], pos, mask=m)
        return c

    lax.fori_loop(0, CH // 16, scan_body, 0)

    pltpu.sync_copy(fp, out_hbm.at[wid])


@functools.partial(
    pl.kernel,
    out_type=jax.ShapeDtypeStruct((NW, 16), jnp.float32),
    mesh=plsc.VectorSubcoreMesh(**_MESH),
    scratch_types=[
        pltpu.VMEM((GPW + 16,), jnp.int32),
        pltpu.VMEM((NW, GPW), jnp.int32),
        pltpu.VMEM((W,), jnp.int32),
        pltpu.VMEM((W,), jnp.float32),
        pltpu.VMEM((W,), jnp.float32),
        pltpu.VMEM((16,), jnp.float32),
        pltpu.VMEM((16,), jnp.float32),
    ],
    compiler_params=pltpu.CompilerParams(needs_layout_passes=False),
)
def _phase_b(fp_hbm, idx_hbm, prd_hbm, tgt_hbm, disc_hbm, out_hbm,
             minbuf, rowbuf, idxw, prdw, tgtw, discb, outb):
    wid = lax.axis_index("s") * NC + lax.axis_index("c")
    col0 = wid * GPW

    pltpu.sync_copy(disc_hbm, discb)
    disc = discb[...]

    pltpu.sync_copy(fp_hbm.at[:, pl.ds(col0, GPW)], rowbuf)

    def min_outer(k, c):
        sl = pl.ds(k * 16, 16)

        def min_inner(r, m):
            return jnp.minimum(m, rowbuf[r, sl])

        minbuf[sl] = lax.fori_loop(1, NW, min_inner, rowbuf[0, sl])
        return c

    lax.fori_loop(0, GPW // 16, min_outer, 0)

    def bf_body(k, carry):
        cv_, cnt = carry
        kk = (GPW // 16 - 1) - k
        sl = pl.ds(kk * 16, 16)
        m = minbuf[sl]
        cnt = cnt + jnp.where(m < NELEM, 1, 0)
        y = lax.rev(m, (0,))
        z = -plsc.cummax(-y)
        sfx = lax.rev(z, (0,))
        out = jnp.minimum(sfx, cv_)
        minbuf[sl] = out
        return (jnp.zeros((16,), jnp.int32) + out[0], cnt)

    _, cntv = lax.fori_loop(
        0, GPW // 16, bf_body,
        (jnp.full((16,), NELEM, jnp.int32), jnp.zeros((16,), jnp.int32)))

    iot = lax.iota(jnp.int32, 16)

    def _pmax(adk, adv, bak, bav):
        t = adk >= bak
        return jnp.where(t, adk, bak), jnp.where(t, adv, bav)

    def _top16_64(ks, vs, final_desc):
        a1k, a1v = plsc.sort_key_val(ks[0], vs[0], descending=False)
        d2k, d2v = plsc.sort_key_val(ks[1], vs[1], descending=True)
        a3k, a3v = plsc.sort_key_val(ks[2], vs[2], descending=False)
        d4k, d4v = plsc.sort_key_val(ks[3], vs[3], descending=True)
        mk, mv = _pmax(d2k, d2v, a1k, a1v)
        t12k, t12v = plsc.sort_key_val(mk, mv, descending=False)
        mk2, mv2 = _pmax(d4k, d4v, a3k, a3v)
        t34k, t34v = plsc.sort_key_val(mk2, mv2, descending=True)
        mk3, mv3 = _pmax(t34k, t34v, t12k, t12v)
        return plsc.sort_key_val(mk3, mv3, descending=final_desc)

    def _top16k_64(ks, final_desc):
        a1 = plsc.sort_key_val(ks[0], ks[0], descending=False)[0]
        d2 = plsc.sort_key_val(ks[1], ks[1], descending=True)[0]
        a3 = plsc.sort_key_val(ks[2], ks[2], descending=False)[0]
        d4 = plsc.sort_key_val(ks[3], ks[3], descending=True)[0]
        m = jnp.maximum(d2, a1)
        t12 = plsc.sort_key_val(m, m, descending=False)[0]
        m2 = jnp.maximum(d4, a3)
        t34 = plsc.sort_key_val(m2, m2, descending=True)[0]
        m3 = jnp.maximum(t34, t12)
        return plsc.sort_key_val(m3, m3, descending=final_desc)[0]

    def ensure(b, p):
        need = jnp.logical_or(p < b, p + 64 > b + W)

        def reload(_):
            nb = (p // 8) * 8
            pltpu.sync_copy(idx_hbm.at[pl.ds(nb, W)], idxw)
            pltpu.sync_copy(prd_hbm.at[pl.ds(nb, W)], prdw)
            pltpu.sync_copy(tgt_hbm.at[pl.ds(nb, W)], tgtw)
            return nb

        return lax.cond(need, reload, lambda _: b, 0)

    def load64(off, g):
        vi = [idxw[pl.ds(off + 16 * q, 16)] for q in range(4)]
        pv = [prdw[pl.ds(off + 16 * q, 16)] for q in range(4)]
        tv = [tgtw[pl.ds(off + 16 * q, 16)] for q in range(4)]
        va = [v == g for v in vi]
        ck = [jnp.where(va[q], pv[q], -1.0) for q in range(4)]
        cv = [jnp.where(va[q], tv[q], 0.0) for q in range(4)]
        ki = [jnp.where(va[q], tv[q], -1.0) for q in range(4)]
        cont = vi[3][15] == g
        return ck, cv, ki, cont

    def group_body(j, carry):
        b, snd = carry
        s = minbuf[pl.ds(j, 16)][0]
        g = col0 + j
        b = ensure(b, s)
        ck, cv, ki, cont = load64(s - b, g)
        ak, av = _top16_64(ck, cv, True)
        ai = _top16k_64(ki, True)

        def cond_fn(st):
            return st[0]

        def body_fn(st):
            _, p, b2, ak2, av2, ai2 = st
            b2 = ensure(b2, p)
            ck2, cv2, ki2, cont2 = load64(p - b2, g)
            xk, xv = _top16_64(ck2, cv2, False)
            mk, mv = _pmax(ak2, av2, xk, xv)
            ak2, av2 = plsc.sort_key_val(mk, mv, descending=True)
            xi = _top16k_64(ki2, False)
            mi = jnp.maximum(ai2, xi)
            ai2 = plsc.sort_key_val(mi, mi, descending=True)[0]
            return (cont2, p + 64, b2, ak2, av2, ai2)

        st = lax.while_loop(cond_fn, body_fn, (cont, s + 64, b, ak, av, ai))
        _, _, b, ak, av, ai = st
        dcg_v = lax.cumsum(av * disc, axis=0)
        idcg_v = lax.cumsum(jnp.maximum(ai, 0.0) * disc, axis=0)
        safe_v = jnp.where(idcg_v > 0.0, idcg_v, 1.0)
        nd_v = jnp.where(idcg_v > 0.0, dcg_v / safe_v, 0.0)
        return (b, snd + nd_v[15])

    base0 = jnp.int32(-2 * W)
    _, snd = lax.fori_loop(0, GPW, group_body, (base0, jnp.float32(0.0)))

    prs = lax.cumsum(cntv, axis=0)[15].astype(jnp.float32)
    outb[...] = jnp.where(iot == 0, snd, jnp.where(iot == 1, prs, 0.0))
    pltpu.sync_copy(outb, out_hbm.at[wid])


_BCE_ROWS = 6400
_BCE_BR = 256


def _bce_body(p_ref, t_ref, o_ref):
    i = pl.program_id(0)
    eps = 1e-7
    p = jnp.clip(p_ref[...], eps, 1.0 - eps)
    t = t_ref[...]
    s = jnp.sum(t * jnp.log(p) + (1.0 - t) * jnp.log(1.0 - p))

    @pl.when(i == 0)
    def _():
        o_ref[0, 0] = s

    @pl.when(i > 0)
    def _():
        o_ref[0, 0] = o_ref[0, 0] + s


def _bce_tc(p2, t2):
    return pl.pallas_call(
        _bce_body,
        grid=(_BCE_ROWS // _BCE_BR,),
        in_specs=[
            pl.BlockSpec((_BCE_BR, 128), lambda i: (i, 0)),
            pl.BlockSpec((_BCE_BR, 128), lambda i: (i, 0)),
        ],
        out_specs=pl.BlockSpec(memory_space=pltpu.SMEM),
        out_shape=jax.ShapeDtypeStruct((1, 1), jnp.float32),
    )(p2, t2)


@jax.jit
def kernel(predictions, targets, indexes):
    idx = indexes.astype(jnp.int32)
    idx_lead = jnp.concatenate([jnp.full((8,), -1, jnp.int32), idx])
    fp = _phase_a(idx_lead)
    idx_pad = jnp.concatenate([idx, jnp.full((PADW,), NQ, jnp.int32)])
    prd_pad = jnp.concatenate([predictions, jnp.zeros((PADW,), jnp.float32)])
    tgt_pad = jnp.concatenate([targets, jnp.zeros((PADW,), jnp.float32)])
    parts = _phase_b(fp, idx_pad, prd_pad, tgt_pad, jnp.asarray(_DISC))
    bce = _bce_tc(predictions.reshape(_BCE_ROWS, 128),
                  targets.reshape(_BCE_ROWS, 128))[0, 0]
    xe = -bce / NELEM
    snd = jnp.sum(parts[:, 0])
    prs = jnp.sum(parts[:, 1])
    ndcg = snd / jnp.maximum(prs, 1.0)
    return xe * (1.0 - ndcg)
